# baseline (speedup 1.0000x reference)
import jax
import jax.numpy as jnp
from jax import lax
from jax.experimental import pallas as pl
from jax.experimental.pallas import tpu as pltpu
from jax.experimental.pallas import tpu_sc as plsc

VOCAB = 1000000
DIM = 64
DIMP = 128
BATCH = 16384

NC = 2
NS = 16
L = 16
NW = NC * NS
B_PER_W = BATCH // NW
CH = 128
NCH = B_PER_W // CH
GPC = CH // L


def _sc_body(cidx_hbm, xidx_hbm, ctr_tbl, ctx_tbl, out_hbm,
             civ, xiv, out_v, part,
             cbuf0, cbuf1, xbuf0, xbuf1, sems):
    wid = lax.axis_index("s") * NC + lax.axis_index("c")
    base = wid * B_PER_W

    pltpu.sync_copy(cidx_hbm.at[pl.ds(base, B_PER_W)], civ)
    pltpu.sync_copy(xidx_hbm.at[pl.ds(base, B_PER_W)], xiv)

    cbufs = (cbuf0, cbuf1)
    xbufs = (xbuf0, xbuf1)

    def copies(ch, b):
        csl = civ.at[pl.ds(ch * CH, CH)]
        xsl = xiv.at[pl.ds(ch * CH, CH)]
        return (pltpu.make_async_copy(ctr_tbl.at[csl], cbufs[b], sems.at[b]),
                pltpu.make_async_copy(ctx_tbl.at[xsl], xbufs[b],
                                      sems.at[2 + b]))

    for b in range(2):
        for cp in copies(b, b):
            cp.start()

    col0 = lax.iota(jnp.int32, L) * L

    for ch in range(NCH):
        b = ch % 2
        for cp in copies(ch, b):
            cp.wait()

        def group(g, carry, b=b, ch=ch):
            for r in range(L):
                row = g * L + r
                p = (cbufs[b][row, pl.ds(0, L)] *
                     xbufs[b][row, pl.ds(0, L)])
                for j in range(1, DIM // L):
                    sl = pl.ds(j * L, L)
                    p = p + cbufs[b][row, sl] * xbufs[b][row, sl]
                part[pl.ds(r * L, L)] = p
            acc = plsc.load_gather(part, [col0])
            for c in range(1, L):
                acc = acc + plsc.load_gather(part, [col0 + c])
            out_v[pl.ds(ch * CH + g * L, L)] = acc
            return carry

        lax.fori_loop(0, GPC, group, 0)

        if ch + 2 < NCH:
            for cp in copies(ch + 2, b):
                cp.start()

    pltpu.sync_copy(out_v, out_hbm.at[pl.ds(base, B_PER_W)])


@jax.jit
def _scores(cidx, xidx, ctr_tbl, ctx_tbl):
    mesh = plsc.VectorSubcoreMesh(
        core_axis_name="c", subcore_axis_name="s",
        num_cores=NC, num_subcores=NS)
    return pl.kernel(
        _sc_body,
        out_type=jax.ShapeDtypeStruct((BATCH,), jnp.float32),
        mesh=mesh,
        scratch_types=[
            pltpu.VMEM((B_PER_W,), jnp.int32),
            pltpu.VMEM((B_PER_W,), jnp.int32),
            pltpu.VMEM((B_PER_W,), jnp.float32),
            pltpu.VMEM((L * L,), jnp.float32),
            pltpu.VMEM((CH, DIMP), jnp.float32),
            pltpu.VMEM((CH, DIMP), jnp.float32),
            pltpu.VMEM((CH, DIMP), jnp.float32),
            pltpu.VMEM((CH, DIMP), jnp.float32),
            pltpu.SemaphoreType.DMA((4,)),
        ],
        compiler_params=pltpu.CompilerParams(
            needs_layout_passes=False, use_tc_tiling_on_sc=True),
    )(cidx, xidx, ctr_tbl, ctx_tbl)


def kernel(center_words, context_words, center_table, context_table):
    cidx = center_words.astype(jnp.int32)
    xidx = context_words.astype(jnp.int32)
    sel = jnp.concatenate(
        [jnp.eye(DIM, dtype=jnp.float32),
         jnp.zeros((DIM, DIMP - DIM), jnp.float32)], axis=1)
    ctr_p = jax.lax.dot(center_table, sel,
                        precision=jax.lax.Precision.HIGHEST)
    ctx_p = jnp.pad(context_table, ((0, 0), (0, DIMP - DIM)))
    return _scores(cidx, xidx, ctr_p, ctx_p)


# speedup vs baseline: 1.6333x; 1.6333x over previous
import jax
import jax.numpy as jnp
from jax import lax
from jax.experimental import pallas as pl
from jax.experimental.pallas import tpu as pltpu
from jax.experimental.pallas import tpu_sc as plsc

VOCAB = 1000000
DIM = 64
BATCH = 16384

NC = 2
NS = 16
L = 16
NW = NC * NS
B_PER_W = BATCH // NW
GROUPS = B_PER_W // L


def _sc_body(cidx_hbm, xidx_hbm, ctr_tbl, ctx_tbl, out_hbm,
             civ, xiv, out_v, part,
             cbuf0, cbuf1, xbuf0, xbuf1, sems):
    wid = lax.axis_index("s") * NC + lax.axis_index("c")
    base = wid * B_PER_W

    pltpu.sync_copy(cidx_hbm.at[pl.ds(base, B_PER_W)], civ)
    pltpu.sync_copy(xidx_hbm.at[pl.ds(base, B_PER_W)], xiv)

    cbufs = (cbuf0, cbuf1)
    xbufs = (xbuf0, xbuf1)

    def fire(g, b):
        cv = civ[pl.ds(g * L, L)]
        xv = xiv[pl.ds(g * L, L)]
        for r in range(L):
            pltpu.make_async_copy(
                ctr_tbl.at[cv[r]], cbufs[b].at[r], sems.at[b]).start()
            pltpu.make_async_copy(
                ctx_tbl.at[xv[r]], xbufs[b].at[r], sems.at[2 + b]).start()

    def drain(b):
        for r in range(L):
            pltpu.make_async_copy(
                ctr_tbl.at[0], cbufs[b].at[r], sems.at[b]).wait()
            pltpu.make_async_copy(
                ctx_tbl.at[0], xbufs[b].at[r], sems.at[2 + b]).wait()

    fire(0, 0)
    fire(1, 1)

    col0 = lax.iota(jnp.int32, L) * L

    def group(g, carry):
        for b in range(2):
            ch = g * 2 + b
            drain(b)
            for r in range(L):
                p = cbufs[b][r, pl.ds(0, L)] * xbufs[b][r, pl.ds(0, L)]
                for j in range(1, DIM // L):
                    sl = pl.ds(j * L, L)
                    p = p + cbufs[b][r, sl] * xbufs[b][r, sl]
                part[pl.ds(r * L, L)] = p
            acc = plsc.load_gather(part, [col0])
            for c in range(1, L):
                acc = acc + plsc.load_gather(part, [col0 + c])
            out_v[pl.ds(ch * L, L)] = acc

            @pl.when(ch + 2 < GROUPS)
            def _():
                fire(ch + 2, b)
        return carry

    lax.fori_loop(0, GROUPS // 2, group, 0)

    pltpu.sync_copy(out_v, out_hbm.at[pl.ds(base, B_PER_W)])


@jax.jit
def _scores(cidx, xidx, ctr_tbl, ctx_tbl):
    mesh = plsc.VectorSubcoreMesh(
        core_axis_name="c", subcore_axis_name="s",
        num_cores=NC, num_subcores=NS)
    return pl.kernel(
        _sc_body,
        out_type=jax.ShapeDtypeStruct((BATCH,), jnp.float32),
        mesh=mesh,
        scratch_types=[
            pltpu.VMEM((B_PER_W,), jnp.int32),
            pltpu.VMEM((B_PER_W,), jnp.int32),
            pltpu.VMEM((B_PER_W,), jnp.float32),
            pltpu.VMEM((L * L,), jnp.float32),
            pltpu.VMEM((L, DIM), jnp.float32),
            pltpu.VMEM((L, DIM), jnp.float32),
            pltpu.VMEM((L, DIM), jnp.float32),
            pltpu.VMEM((L, DIM), jnp.float32),
            pltpu.SemaphoreType.DMA((4,)),
        ],
        compiler_params=pltpu.CompilerParams(
            needs_layout_passes=False, use_tc_tiling_on_sc=True),
    )(cidx, xidx, ctr_tbl, ctx_tbl)


def kernel(center_words, context_words, center_table, context_table):
    cidx = center_words.astype(jnp.int32)
    xidx = context_words.astype(jnp.int32)
    return _scores(cidx, xidx, center_table, context_table)
